# baseline (speedup 1.0000x reference)
import functools

import jax
import jax.numpy as jnp
from jax import lax
from jax.experimental import pallas as pl
from jax.experimental.pallas import tpu as pltpu
from jax.experimental.pallas import tpu_sc as plsc

ALPHA = 0.2
NC = 2
NS = 16
L = 16


def _prep_body(feat_ref, w_ref, a_ref, msg2_ref, as_ref, ad_ref, m_ref):
    d = w_ref.shape[0]
    d2 = d // 2
    feat = feat_ref[...]
    msg = jnp.dot(feat, w_ref[...], preferred_element_type=jnp.float32)
    msg2_ref[...] = msg
    a_all = a_ref[...]
    dn = (((0,), (1,)), ((), ()))
    asv = lax.dot_general(a_all[:d, :], msg, dn,
                          preferred_element_type=jnp.float32)
    adv = lax.dot_general(a_all[d:, :], msg, dn,
                          preferred_element_type=jnp.float32)
    as_ref[...] = asv
    ad_ref[...] = adv
    mraw = jnp.max(asv) + jnp.max(adv)
    m = jnp.where(mraw > 0, mraw, ALPHA * mraw)
    m_ref[...] = jnp.full((1, L), m, dtype=jnp.float32)


def _prep(feat, W, a):
    n, d = feat.shape
    return pl.pallas_call(
        _prep_body,
        out_shape=(
            jax.ShapeDtypeStruct((n, d), jnp.float32),
            jax.ShapeDtypeStruct((1, n), jnp.float32),
            jax.ShapeDtypeStruct((1, n), jnp.float32),
            jax.ShapeDtypeStruct((1, L), jnp.float32),
        ),
    )(feat, W, a)


def _edge_kernel_body(nch, k, n, d2,
                      edges_hbm, as_hbm, ad_hbm, m_hbm, msg2_hbm,
                      num_hbm,
                      src_v, dst_v, as_v, ad_v, m_v, g0, g1, s0, s1, eb0, eb1,
                      num_sh, den_sh, gsem0, gsem1, ssem0, ssem1):
    cid = lax.axis_index("c")
    sid = lax.axis_index("s")

    pltpu.sync_copy(edges_hbm.at[0, sid], src_v)
    pltpu.sync_copy(edges_hbm.at[1, sid], dst_v)
    pltpu.sync_copy(as_hbm.at[0], as_v)
    pltpu.sync_copy(ad_hbm.at[0], ad_v)
    pltpu.sync_copy(m_hbm, m_v)

    zeros16 = jnp.zeros((L,), jnp.float32)

    def _zrow_body(i, carry):
        for j in range(d2 // L):
            s0[i, pl.ds(j * L, L)] = zeros16
        return carry

    lax.fori_loop(0, k, _zrow_body, 0)

    def _zden_body(i, carry):
        eb0[pl.ds(i * L, L)] = zeros16
        return carry

    lax.fori_loop(0, k // L, _zden_body, 0)

    nzc = n // k
    per = (nzc + NS - 1) // NS

    def _zero_chunk(ci, carry):
        pltpu.sync_copy(s0, num_sh.at[pl.ds(ci * k, k)])
        pltpu.sync_copy(eb0, den_sh.at[pl.ds(ci * k, k)])
        return carry

    lax.fori_loop(sid * per, jnp.minimum((sid + 1) * per, nzc),
                  _zero_chunk, 0)
    plsc.subcore_barrier()

    def _xform_body(i, carry):
        for g in range(k // L):
            v = src_v[i, pl.ds(g * L, L)]
            src_v[i, pl.ds(g * L, L)] = v + v + cid
        return carry

    lax.fori_loop(0, nch, _xform_body, 0)

    mvec = m_v[0, :]
    cidv = jnp.full((L,), cid, dtype=jnp.int32)
    msg_c = msg2_hbm

    def _compute_chunk(c, gbuf, sbuf, eb):
        for g in range(k // L):
            sidx = lax.shift_right_logical(src_v[c, pl.ds(g * L, L)] - cidv,
                                           1)
            didx = dst_v[c, pl.ds(g * L, L)]
            s = plsc.load_gather(as_v, [sidx]) + plsc.load_gather(ad_v, [didx])
            s = jnp.where(s > 0, s, ALPHA * s)
            ev = jnp.exp(s - mvec)
            eb[pl.ds(g * L, L)] = ev
            for l in range(L):
                row = g * L + l
                coef = jnp.full((L,), ev[l], dtype=jnp.float32)
                for j in range(d2 // (2 * L)):
                    xi = gbuf[row, pl.ds(j * L, L)]
                    lo = plsc.bitcast(lax.shift_left(xi, 16), jnp.float32)
                    hi = plsc.bitcast(
                        lax.bitwise_and(xi, jnp.int32(-65536)), jnp.float32)
                    sbuf[row, pl.ds(j * 2 * L, L)] = lo * coef
                    sbuf[row, pl.ds(j * 2 * L + L, L)] = hi * coef

    def _issue_gather(c, gbuf, gsem):
        pltpu.async_copy(msg_c.at[src_v.at[c]], gbuf, gsem)

    def _wait_gather(c, gbuf, gsem):
        pltpu.make_async_copy(msg_c.at[src_v.at[c]], gbuf, gsem).wait()

    def _issue_scatter(c, sbuf, eb, ssem):
        pltpu.async_copy(sbuf, num_sh.at[dst_v.at[c]], ssem, add=True)
        pltpu.async_copy(eb, den_sh.at[dst_v.at[c]], ssem, add=True)

    def _wait_scatter(sbuf, eb, ssem):
        pltpu.make_async_copy(sbuf, num_sh.at[dst_v.at[0]], ssem).wait()
        pltpu.make_async_copy(eb, den_sh.at[dst_v.at[0]], ssem).wait()

    _issue_gather(0, g0, gsem0)

    def _step(c, gbuf, gsem, sbuf, eb, ssem, other_g, other_gsem):
        @pl.when(c + 1 < nch)
        def _():
            _issue_gather(c + 1, other_g, other_gsem)

        _wait_gather(c, gbuf, gsem)

        @pl.when(c >= 2)
        def _():
            _wait_scatter(sbuf, eb, ssem)

        _compute_chunk(c, gbuf, sbuf, eb)
        _issue_scatter(c, sbuf, eb, ssem)

    def _pair_body(i, carry):
        c0 = 2 * i
        _step(c0, g0, gsem0, s0, eb0, ssem0, g1, gsem1)
        _step(c0 + 1, g1, gsem1, s1, eb1, ssem1, g0, gsem0)
        return carry

    lax.fori_loop(0, nch // 2, _pair_body, 0)
    _wait_scatter(s0, eb0, ssem0)
    _wait_scatter(s1, eb1, ssem1)

    plsc.subcore_barrier()

    def _out_chunk(ci, carry):
        pltpu.sync_copy(num_sh.at[pl.ds(ci * k, k)], s0)
        pltpu.sync_copy(den_sh.at[pl.ds(ci * k, k)], eb0)
        for g in range(k // L):
            dv = eb0[pl.ds(g * L, L)]
            inv = 1.0 / (dv + 1e-10)
            for l in range(L):
                row = g * L + l
                coef = jnp.full((L,), inv[l], dtype=jnp.float32)
                for j in range(d2 // L):
                    s0[row, pl.ds(j * L, L)] = s0[row, pl.ds(j * L, L)] * coef
        pltpu.sync_copy(
            s0, num_hbm.at[pl.ds(ci * k, k), pl.ds(cid * d2, d2)])
        return carry

    lax.fori_loop(sid * per, jnp.minimum((sid + 1) * per, nzc),
                  _out_chunk, 0)


def _edge_pass(edges4, asv, adv, m16, msgv):
    n = msgv.shape[0] // NC
    d2 = msgv.shape[1] * 2
    _, _, nch, k = edges4.shape
    body = functools.partial(_edge_kernel_body, nch, k, n, d2)
    return pl.kernel(
        body,
        out_type=jax.ShapeDtypeStruct((n, NC * d2), jnp.float32),
        mesh=plsc.VectorSubcoreMesh(core_axis_name="c", subcore_axis_name="s",
                                    num_cores=NC, num_subcores=NS),
        compiler_params=pltpu.CompilerParams(use_tc_tiling_on_sc=False,
                                             needs_layout_passes=False),
        scratch_types=[
            pltpu.VMEM((nch, k), jnp.int32),
            pltpu.VMEM((nch, k), jnp.int32),
            pltpu.VMEM((n,), jnp.float32),
            pltpu.VMEM((n,), jnp.float32),
            pltpu.VMEM((1, L), jnp.float32),
            pltpu.VMEM((k, d2 // 2), jnp.int32),
            pltpu.VMEM((k, d2 // 2), jnp.int32),
            pltpu.VMEM((k, d2), jnp.float32),
            pltpu.VMEM((k, d2), jnp.float32),
            pltpu.VMEM((k,), jnp.float32),
            pltpu.VMEM((k,), jnp.float32),
            pltpu.VMEM_SHARED((n, d2), jnp.float32),
            pltpu.VMEM_SHARED((n,), jnp.float32),
            pltpu.SemaphoreType.DMA,
            pltpu.SemaphoreType.DMA,
            pltpu.SemaphoreType.DMA,
            pltpu.SemaphoreType.DMA,
        ],
    )(edges4, asv, adv, m16, msgv)


def _post_body(num_ref, gamma_ref, beta_ref, out_ref):
    o = num_ref[...]
    mean = jnp.mean(o, axis=0, keepdims=True)
    var = jnp.mean((o - mean) * (o - mean), axis=0, keepdims=True)
    inv = lax.rsqrt(var + 1e-5)
    out_ref[...] = (o - mean) * inv * gamma_ref[...] + beta_ref[...]


def _post(num, gamma, beta):
    n = num.shape[0]
    d = gamma.shape[-1]
    return pl.pallas_call(
        _post_body,
        out_shape=jax.ShapeDtypeStruct((n, d), jnp.float32),
    )(num, gamma.reshape(1, d), beta.reshape(1, d))


def kernel(feat, edges, W, a, gamma, beta):
    n, d = feat.shape
    e = edges.shape[1]
    k = 80
    nch = e // (NS * k)
    msg, asv, adv, m16 = _prep(feat, W, a)
    edges4 = edges.astype(jnp.int32).reshape(2, NS, nch, k)
    bits = lax.bitcast_convert_type(msg, jnp.uint32)
    rnd = (bits + jnp.uint32(0x7FFF) + ((bits >> 16) & jnp.uint32(1))) >> 16
    r4 = rnd.reshape(n, d // 32, 2, 16)
    packed = (r4[:, :, 0, :] | (r4[:, :, 1, :] << 16)).astype(jnp.uint32)
    packed = lax.bitcast_convert_type(packed, jnp.int32)
    msgv = packed.reshape(NC * n, d // (2 * NC))
    num = _edge_pass(edges4, asv, adv, m16, msgv)
    return _post(num, gamma, beta)


# speedup vs baseline: 1.1168x; 1.1168x over previous
import functools

import jax
import jax.numpy as jnp
from jax import lax
from jax.experimental import pallas as pl
from jax.experimental.pallas import tpu as pltpu
from jax.experimental.pallas import tpu_sc as plsc

ALPHA = 0.2
NC = 2
NS = 16
L = 16


def _prep_body(feat_ref, w_ref, a_ref, msg2_ref, as_ref, ad_ref, m_ref):
    d = w_ref.shape[0]
    d2 = d // 2
    feat = feat_ref[...]
    msg = jnp.dot(feat, w_ref[...], preferred_element_type=jnp.float32)
    msg2_ref[...] = msg
    a_all = a_ref[...]
    dn = (((0,), (1,)), ((), ()))
    asv = lax.dot_general(a_all[:d, :], msg, dn,
                          preferred_element_type=jnp.float32)
    adv = lax.dot_general(a_all[d:, :], msg, dn,
                          preferred_element_type=jnp.float32)
    as_ref[...] = asv
    ad_ref[...] = adv
    mraw = jnp.max(asv) + jnp.max(adv)
    m = jnp.where(mraw > 0, mraw, ALPHA * mraw)
    m_ref[...] = jnp.full((1, L), m, dtype=jnp.float32)


def _prep(feat, W, a):
    n, d = feat.shape
    return pl.pallas_call(
        _prep_body,
        out_shape=(
            jax.ShapeDtypeStruct((n, d), jnp.float32),
            jax.ShapeDtypeStruct((1, n), jnp.float32),
            jax.ShapeDtypeStruct((1, n), jnp.float32),
            jax.ShapeDtypeStruct((1, L), jnp.float32),
        ),
    )(feat, W, a)


def _edge_kernel_body(nch, k, n, d2,
                      edges_hbm, as_hbm, ad_hbm, m_hbm, msg2_hbm,
                      num_hbm,
                      src_v, dst_v, as_v, ad_v, m_v, g0, g1, s0, s1, eb0, eb1,
                      num_sh, den_sh, gsem0, gsem1, ssem0, ssem1):
    cid = lax.axis_index("c")
    sid = lax.axis_index("s")

    pltpu.sync_copy(edges_hbm.at[0, sid], src_v)
    pltpu.sync_copy(edges_hbm.at[1, sid], dst_v)
    pltpu.sync_copy(as_hbm.at[0], as_v)
    pltpu.sync_copy(ad_hbm.at[0], ad_v)
    pltpu.sync_copy(m_hbm, m_v)

    zeros16 = jnp.zeros((L,), jnp.float32)

    def _zrow_body(i, carry):
        for j in range(d2 // L):
            s0[i, pl.ds(j * L, L)] = zeros16
        return carry

    lax.fori_loop(0, k, _zrow_body, 0)

    def _zden_body(i, carry):
        eb0[pl.ds(i * L, L)] = zeros16
        return carry

    lax.fori_loop(0, k // L, _zden_body, 0)

    nzc = n // k
    per = (nzc + NS - 1) // NS

    def _zero_chunk(ci, carry):
        pltpu.sync_copy(s0, num_sh.at[pl.ds(ci * k, k)])
        pltpu.sync_copy(eb0, den_sh.at[pl.ds(ci * k, k)])
        return carry

    lax.fori_loop(sid * per, jnp.minimum((sid + 1) * per, nzc),
                  _zero_chunk, 0)
    plsc.subcore_barrier()

    def _xform_body(i, carry):
        for g in range(k // L):
            v = src_v[i, pl.ds(g * L, L)]
            src_v[i, pl.ds(g * L, L)] = v + v + cid
        return carry

    lax.fori_loop(0, nch, _xform_body, 0)

    mvec = m_v[0, :]
    cidv = jnp.full((L,), cid, dtype=jnp.int32)
    msg_c = msg2_hbm

    def _compute_chunk(c, gbuf, sbuf, eb):
        for g in range(k // L):
            sidx = lax.shift_right_logical(src_v[c, pl.ds(g * L, L)] - cidv,
                                           1)
            didx = dst_v[c, pl.ds(g * L, L)]
            s = plsc.load_gather(as_v, [sidx]) + plsc.load_gather(ad_v, [didx])
            s = jnp.where(s > 0, s, ALPHA * s)
            ev = jnp.exp(s - mvec)
            eb[pl.ds(g * L, L)] = ev
            for l in range(L):
                row = g * L + l
                coef = jnp.full((L,), ev[l], dtype=jnp.float32)
                for j in range(d2 // L):
                    sbuf[row, pl.ds(j * L, L)] = (
                        gbuf[row, pl.ds(j * L, L)] * coef)

    def _issue_gather(c, gbuf, gsem):
        pltpu.async_copy(msg_c.at[src_v.at[c]], gbuf, gsem)

    def _wait_gather(c, gbuf, gsem):
        pltpu.make_async_copy(msg_c.at[src_v.at[c]], gbuf, gsem).wait()

    def _issue_scatter(c, sbuf, eb, ssem):
        pltpu.async_copy(sbuf, num_sh.at[dst_v.at[c]], ssem, add=True)
        pltpu.async_copy(eb, den_sh.at[dst_v.at[c]], ssem, add=True)

    def _wait_scatter(sbuf, eb, ssem):
        pltpu.make_async_copy(sbuf, num_sh.at[dst_v.at[0]], ssem).wait()
        pltpu.make_async_copy(eb, den_sh.at[dst_v.at[0]], ssem).wait()

    _issue_gather(0, g0, gsem0)

    def _step(c, gbuf, gsem, sbuf, eb, ssem, other_g, other_gsem):
        @pl.when(c + 1 < nch)
        def _():
            _issue_gather(c + 1, other_g, other_gsem)

        _wait_gather(c, gbuf, gsem)

        @pl.when(c >= 2)
        def _():
            _wait_scatter(sbuf, eb, ssem)

        _compute_chunk(c, gbuf, sbuf, eb)
        _issue_scatter(c, sbuf, eb, ssem)

    def _pair_body(i, carry):
        c0 = 2 * i
        _step(c0, g0, gsem0, s0, eb0, ssem0, g1, gsem1)
        _step(c0 + 1, g1, gsem1, s1, eb1, ssem1, g0, gsem0)
        return carry

    lax.fori_loop(0, nch // 2, _pair_body, 0)
    _wait_scatter(s0, eb0, ssem0)
    _wait_scatter(s1, eb1, ssem1)

    plsc.subcore_barrier()

    def _out_chunk(ci, carry):
        pltpu.sync_copy(num_sh.at[pl.ds(ci * k, k)], s0)
        pltpu.sync_copy(den_sh.at[pl.ds(ci * k, k)], eb0)
        for g in range(k // L):
            dv = eb0[pl.ds(g * L, L)]
            inv = 1.0 / (dv + 1e-10)
            for l in range(L):
                row = g * L + l
                coef = jnp.full((L,), inv[l], dtype=jnp.float32)
                for j in range(d2 // L):
                    s0[row, pl.ds(j * L, L)] = s0[row, pl.ds(j * L, L)] * coef
        pltpu.sync_copy(
            s0, num_hbm.at[pl.ds(ci * k, k), pl.ds(cid * d2, d2)])
        return carry

    lax.fori_loop(sid * per, jnp.minimum((sid + 1) * per, nzc),
                  _out_chunk, 0)


def _edge_pass(edges4, asv, adv, m16, msgv):
    n = msgv.shape[0] // NC
    d2 = msgv.shape[1]
    _, _, nch, k = edges4.shape
    body = functools.partial(_edge_kernel_body, nch, k, n, d2)
    return pl.kernel(
        body,
        out_type=jax.ShapeDtypeStruct((n, NC * d2), jnp.float32),
        mesh=plsc.VectorSubcoreMesh(core_axis_name="c", subcore_axis_name="s",
                                    num_cores=NC, num_subcores=NS),
        compiler_params=pltpu.CompilerParams(use_tc_tiling_on_sc=False,
                                             needs_layout_passes=False),
        scratch_types=[
            pltpu.VMEM((nch, k), jnp.int32),
            pltpu.VMEM((nch, k), jnp.int32),
            pltpu.VMEM((n,), jnp.float32),
            pltpu.VMEM((n,), jnp.float32),
            pltpu.VMEM((1, L), jnp.float32),
            pltpu.VMEM((k, d2), jnp.float32),
            pltpu.VMEM((k, d2), jnp.float32),
            pltpu.VMEM((k, d2), jnp.float32),
            pltpu.VMEM((k, d2), jnp.float32),
            pltpu.VMEM((k,), jnp.float32),
            pltpu.VMEM((k,), jnp.float32),
            pltpu.VMEM_SHARED((n, d2), jnp.float32),
            pltpu.VMEM_SHARED((n,), jnp.float32),
            pltpu.SemaphoreType.DMA,
            pltpu.SemaphoreType.DMA,
            pltpu.SemaphoreType.DMA,
            pltpu.SemaphoreType.DMA,
        ],
    )(edges4, asv, adv, m16, msgv)


def _post_body(num_ref, gamma_ref, beta_ref, out_ref):
    o = num_ref[...]
    mean = jnp.mean(o, axis=0, keepdims=True)
    var = jnp.mean((o - mean) * (o - mean), axis=0, keepdims=True)
    inv = lax.rsqrt(var + 1e-5)
    out_ref[...] = (o - mean) * inv * gamma_ref[...] + beta_ref[...]


def _post(num, gamma, beta):
    n = num.shape[0]
    d = gamma.shape[-1]
    return pl.pallas_call(
        _post_body,
        out_shape=jax.ShapeDtypeStruct((n, d), jnp.float32),
    )(num, gamma.reshape(1, d), beta.reshape(1, d))


def kernel(feat, edges, W, a, gamma, beta):
    n, d = feat.shape
    e = edges.shape[1]
    k = 80
    nch = e // (NS * k)
    msg, asv, adv, m16 = _prep(feat, W, a)
    edges4 = edges.astype(jnp.int32).reshape(2, NS, nch, k)
    num = _edge_pass(edges4, asv, adv, m16, msg.reshape(NC * n, d // NC))
    return _post(num, gamma, beta)
